# baseline (speedup 1.0000x reference)
import functools

import jax
import jax.numpy as jnp
from jax import lax
from jax.experimental import pallas as pl
from jax.experimental.pallas import tpu as pltpu
from jax.experimental.pallas import tpu_sc as plsc

B = 4096
L = 200
D = 64
V = 1000000
NC = 2
NS = 16
NW = NC * NS
RPT = B // NW
NBUF = 2
NFV = 192 // 16

GV = 256
VMAIN = 999424
GPT = VMAIN // GV // NW
NXG = (V // GV) - GPT * NW
VTAIL = V - (V // GV) * GV


def _fmt_body(tabt_hbm, tail_hbm, lin_hbm, tin, tout0, tout1,
              si0, si1, so0, so1):
    wid = lax.axis_index("s") * NC + lax.axis_index("c")
    touts = (tout0, tout1)
    sis = (si0, si1)
    sos = (so0, so1)

    def issue_in(k, g):
        voc0 = g * GV
        for fg in range(8):
            pltpu.async_copy(
                tabt_hbm.at[pl.ds(fg * 8, 8), pl.ds(voc0, GV)],
                tin.at[k, fg], sis[k])

    def drain_in(k, g):
        voc0 = g * GV
        for fg in range(8):
            pltpu.make_async_copy(
                tabt_hbm.at[pl.ds(fg * 8, 8), pl.ds(voc0, GV)],
                tin.at[k, fg], sis[k]).wait()

    def issue_out(k, g):
        pltpu.async_copy(touts[k], lin_hbm.at[pl.ds(g * GV * D, GV * D)],
                         sos[k])

    def drain_out(k, g):
        pltpu.make_async_copy(
            touts[k], lin_hbm.at[pl.ds(g * GV * D, GV * D)], sos[k]).wait()

    lanes64 = lax.iota(jnp.int32, 16) * D

    def transpose_group(k):
        @plsc.parallel_loop(0, GV, step=16)
        def _(c0):
            idx0 = lanes64 + c0 * D
            for fg in range(8):
                for fr in range(8):
                    v = tin[k, fg, fr, pl.ds(c0, 16)]
                    plsc.store_scatter(touts[k], [idx0 + (fg * 8 + fr)], v)

    def my_group(w, j):
        return w * GPT + j

    issue_in(0, my_group(wid, 0))
    issue_in(1, my_group(wid, 1))

    @pl.loop(0, GPT, step=2)
    def _(j):
        for k in range(2):
            g = my_group(wid, j + k)
            drain_in(k, g)

            @pl.when(j + k >= 2)
            def _():
                drain_out(k, my_group(wid, j + k - 2))

            transpose_group(k)
            issue_out(k, g)

            @pl.when(j + k + 2 < GPT)
            def _():
                issue_in(k, my_group(wid, j + k + 2))

    drain_out(0, my_group(wid, GPT - 2))
    drain_out(1, my_group(wid, GPT - 1))

    @pl.when(wid < NXG)
    def _():
        g = NW * GPT + wid
        issue_in(0, g)
        drain_in(0, g)
        transpose_group(0)
        issue_out(0, g)
        drain_out(0, g)

    @pl.when(wid == NXG)
    def _():
        pltpu.async_copy(tail_hbm, tout0.at[pl.ds(0, VTAIL * D)], si0)
        pltpu.make_async_copy(
            tail_hbm, tout0.at[pl.ds(0, VTAIL * D)], si0).wait()
        pltpu.async_copy(tout0.at[pl.ds(0, VTAIL * D)],
                         lin_hbm.at[pl.ds((V - VTAIL) * D, VTAIL * D)], so0)
        pltpu.make_async_copy(
            tout0.at[pl.ds(0, VTAIL * D)],
            lin_hbm.at[pl.ds((V - VTAIL) * D, VTAIL * D)], so0).wait()


def _tile_body(x_hbm, tab_hbm, out_hbm, x_v, bufs, out_v, *sems):
    wid = lax.axis_index("s") * NC + lax.axis_index("c")
    base = wid * RPT

    pltpu.sync_copy(x_hbm.at[pl.ds(base * L, RPT * L)], x_v)

    def issue(k, r):
        pltpu.async_copy(
            tab_hbm.at[x_v.at[pl.ds(r * L, L)]], bufs.at[k], sems[k])

    def drain(k, r):
        pltpu.make_async_copy(
            tab_hbm.at[x_v.at[pl.ds(r * L, L)]], bufs.at[k], sems[k]).wait()

    for k in range(NBUF):
        issue(k, k)

    lanes = lax.iota(jnp.int32, 16)

    @pl.loop(0, RPT, step=NBUF)
    def _(g):
        for k in range(NBUF):
            r = g + k
            drain(k, r)

            cnt = jnp.zeros((16,), jnp.int32)
            for c in range(NFV):
                v = x_v[pl.ds(r * L + c * 16, 16)]
                cnt = cnt + plsc.all_reduce_population_count(v != 0)
            vt = x_v[pl.ds(r * L + 184, 16)]
            cnt = cnt + plsc.all_reduce_population_count(
                (vt != 0) & (lanes >= 8))
            denom = jnp.maximum(cnt.astype(jnp.float32),
                                jnp.full((16,), 1.0, jnp.float32))

            zero = jnp.zeros((16,), jnp.float32)

            @plsc.parallel_loop(0, L, unroll=8, carry=(zero,) * 4)
            def acc(l, a):
                return tuple(a[d] + bufs[k, l, pl.ds(d * 16, 16)]
                             for d in range(4))

            for d in range(4):
                out_v[r, pl.ds(d * 16, 16)] = acc[d] / denom

            @pl.when(r + NBUF < RPT)
            def _():
                issue(k, r + NBUF)

    pltpu.sync_copy(out_v, out_hbm.at[pl.ds(base, RPT)])


@jax.jit
def _run(x_flat, table):
    mesh = plsc.VectorSubcoreMesh(core_axis_name="c", subcore_axis_name="s")

    tabt = jnp.transpose(table)
    tail = jnp.reshape(table[V - VTAIL:], (-1,))

    fmt_kernel = functools.partial(
        pl.kernel,
        out_type=jax.ShapeDtypeStruct((V * D,), jnp.float32),
        mesh=mesh,
        compiler_params=pltpu.CompilerParams(use_tc_tiling_on_sc=True,
                                             needs_layout_passes=False),
        scratch_types=[
            pltpu.VMEM((2, 8, 8, GV), jnp.float32),
            pltpu.VMEM((GV * D,), jnp.float32),
            pltpu.VMEM((GV * D,), jnp.float32),
        ] + [pltpu.SemaphoreType.DMA] * 4,
    )(_fmt_body)
    tab_lin = fmt_kernel(tabt, tail)

    grid_kernel = functools.partial(
        pl.kernel,
        out_type=jax.ShapeDtypeStruct((B, D), jnp.float32),
        mesh=mesh,
        compiler_params=pltpu.CompilerParams(use_tc_tiling_on_sc=False,
                                             needs_layout_passes=False),
        scratch_types=[
            pltpu.VMEM((RPT * L,), jnp.int32),
            pltpu.VMEM((NBUF, L, D), jnp.float32),
            pltpu.VMEM((RPT, D), jnp.float32),
        ] + [pltpu.SemaphoreType.DMA] * NBUF,
    )(_tile_body)
    return grid_kernel(x_flat, jnp.reshape(tab_lin, (V, D)))


def kernel(x, table):
    return _run(jnp.reshape(x.astype(jnp.int32), (-1,)), table)


# speedup vs baseline: 2.1871x; 2.1871x over previous
import functools

import jax
import jax.numpy as jnp
from jax import lax
from jax.experimental import pallas as pl
from jax.experimental.pallas import tpu as pltpu
from jax.experimental.pallas import tpu_sc as plsc

B = 4096
L = 200
D = 64
V = 1000000
NC = 2
NS = 16
NW = NC * NS
RPT = B // NW
NBUF = 2
NFV = 192 // 16

GV = 128
GPT = (V // GV) // NW
NXG = (V // GV) - GPT * NW
VTAIL = V - (V // GV) * GV


def _fmt_body(tabt_hbm, tail_hbm, lin_hbm, tin, tout0, tout1,
              si0, si1, so0, so1):
    wid = lax.axis_index("s") * NC + lax.axis_index("c")
    touts = (tout0, tout1)
    sis = (si0, si1)
    sos = (so0, so1)

    def issue_in(k, g):
        voc0 = g * GV
        for fg in range(8):
            pltpu.async_copy(
                tabt_hbm.at[pl.ds(fg * 8, 8), pl.ds(voc0, GV)],
                tin.at[k, fg], sis[k])

    def drain_in(k, g):
        voc0 = g * GV
        for fg in range(8):
            pltpu.make_async_copy(
                tabt_hbm.at[pl.ds(fg * 8, 8), pl.ds(voc0, GV)],
                tin.at[k, fg], sis[k]).wait()

    def issue_out(k, g):
        pltpu.async_copy(touts[k], lin_hbm.at[pl.ds(g * GV * D, GV * D)],
                         sos[k])

    def drain_out(k, g):
        pltpu.make_async_copy(
            touts[k], lin_hbm.at[pl.ds(g * GV * D, GV * D)], sos[k]).wait()

    lanes = lax.iota(jnp.int32, 16)

    def transpose_group(k):
        ms = [(lanes + j) & 15 for j in range(16)]
        ihis = [m >> 3 for m in ms]
        ilos = [m & 7 for m in ms]
        sps = [lanes * D + m for m in ms]

        @plsc.parallel_loop(0, GV, step=16)
        def _(v0):
            iv = lanes + v0
            for f0 in range(0, D, 16):
                for j in range(16):
                    v = plsc.load_gather(
                        tin.at[k], [ihis[j] + (f0 >> 3), ilos[j], iv])
                    plsc.store_scatter(
                        touts[k], [sps[j] + (v0 * D + f0)], v)

    def my_group(w, j):
        return w * GPT + j

    issue_in(0, my_group(wid, 0))
    issue_in(1, my_group(wid, 1))

    @pl.loop(0, GPT, step=2)
    def _(j):
        for k in range(2):
            g = my_group(wid, j + k)
            drain_in(k, g)

            @pl.when(j + k >= 2)
            def _():
                drain_out(k, my_group(wid, j + k - 2))

            transpose_group(k)
            issue_out(k, g)

            @pl.when(j + k + 2 < GPT)
            def _():
                issue_in(k, my_group(wid, j + k + 2))

    drain_out(0, my_group(wid, GPT - 2))
    drain_out(1, my_group(wid, GPT - 1))

    @pl.when(wid < NXG)
    def _():
        g = NW * GPT + wid
        issue_in(0, g)
        drain_in(0, g)
        transpose_group(0)
        issue_out(0, g)
        drain_out(0, g)

    @pl.when(wid == NXG)
    def _():
        pltpu.async_copy(tail_hbm, tout0.at[pl.ds(0, VTAIL * D)], si0)
        pltpu.make_async_copy(
            tail_hbm, tout0.at[pl.ds(0, VTAIL * D)], si0).wait()
        pltpu.async_copy(tout0.at[pl.ds(0, VTAIL * D)],
                         lin_hbm.at[pl.ds((V - VTAIL) * D, VTAIL * D)], so0)
        pltpu.make_async_copy(
            tout0.at[pl.ds(0, VTAIL * D)],
            lin_hbm.at[pl.ds((V - VTAIL) * D, VTAIL * D)], so0).wait()


def _tile_body(x_hbm, tab_hbm, out_hbm, x_v, bufs, out_v, *sems):
    wid = lax.axis_index("s") * NC + lax.axis_index("c")
    base = wid * RPT

    pltpu.sync_copy(x_hbm.at[pl.ds(base * L, RPT * L)], x_v)

    def issue(k, r):
        pltpu.async_copy(
            tab_hbm.at[x_v.at[pl.ds(r * L, L)]], bufs.at[k], sems[k])

    def drain(k, r):
        pltpu.make_async_copy(
            tab_hbm.at[x_v.at[pl.ds(r * L, L)]], bufs.at[k], sems[k]).wait()

    for k in range(NBUF):
        issue(k, k)

    lanes = lax.iota(jnp.int32, 16)

    @pl.loop(0, RPT, step=NBUF)
    def _(g):
        for k in range(NBUF):
            r = g + k
            drain(k, r)

            cnt = jnp.zeros((16,), jnp.int32)
            for c in range(NFV):
                v = x_v[pl.ds(r * L + c * 16, 16)]
                cnt = cnt + plsc.all_reduce_population_count(v != 0)
            vt = x_v[pl.ds(r * L + 184, 16)]
            cnt = cnt + plsc.all_reduce_population_count(
                (vt != 0) & (lanes >= 8))
            denom = jnp.maximum(cnt.astype(jnp.float32),
                                jnp.full((16,), 1.0, jnp.float32))

            zero = jnp.zeros((16,), jnp.float32)

            @plsc.parallel_loop(0, L, unroll=8, carry=(zero,) * 4)
            def acc(l, a):
                return tuple(a[d] + bufs[k, l, pl.ds(d * 16, 16)]
                             for d in range(4))

            for d in range(4):
                out_v[r, pl.ds(d * 16, 16)] = acc[d] / denom

            @pl.when(r + NBUF < RPT)
            def _():
                issue(k, r + NBUF)

    pltpu.sync_copy(out_v, out_hbm.at[pl.ds(base, RPT)])


@jax.jit
def _run(x_flat, table):
    mesh = plsc.VectorSubcoreMesh(core_axis_name="c", subcore_axis_name="s")

    tabt = jnp.transpose(table)
    tail = jnp.reshape(table[V - VTAIL:], (-1,))

    fmt_kernel = functools.partial(
        pl.kernel,
        out_type=jax.ShapeDtypeStruct((V * D,), jnp.float32),
        mesh=mesh,
        compiler_params=pltpu.CompilerParams(use_tc_tiling_on_sc=True,
                                             needs_layout_passes=False),
        scratch_types=[
            pltpu.VMEM((2, 8, 8, GV), jnp.float32),
            pltpu.VMEM((GV * D,), jnp.float32),
            pltpu.VMEM((GV * D,), jnp.float32),
        ] + [pltpu.SemaphoreType.DMA] * 4,
    )(_fmt_body)
    tab_lin = fmt_kernel(tabt, tail)

    grid_kernel = functools.partial(
        pl.kernel,
        out_type=jax.ShapeDtypeStruct((B, D), jnp.float32),
        mesh=mesh,
        compiler_params=pltpu.CompilerParams(use_tc_tiling_on_sc=False,
                                             needs_layout_passes=False),
        scratch_types=[
            pltpu.VMEM((RPT * L,), jnp.int32),
            pltpu.VMEM((NBUF, L, D), jnp.float32),
            pltpu.VMEM((RPT, D), jnp.float32),
        ] + [pltpu.SemaphoreType.DMA] * NBUF,
    )(_tile_body)
    return grid_kernel(x_flat, jnp.reshape(tab_lin, (V, D)))


def kernel(x, table):
    return _run(jnp.reshape(x.astype(jnp.int32), (-1,)), table)


# speedup vs baseline: 2.3746x; 1.0857x over previous
import functools

import jax
import jax.numpy as jnp
from jax import lax
from jax.experimental import pallas as pl
from jax.experimental.pallas import tpu as pltpu
from jax.experimental.pallas import tpu_sc as plsc

B = 4096
L = 200
D = 64
V = 1000000
NC = 2
NS = 16
NW = NC * NS
RPT = B // NW
NBUF = 4
NFV = 192 // 16

GV = 128
GPT = (V // GV) // NW
NXG = (V // GV) - GPT * NW
VTAIL = V - (V // GV) * GV


def _fmt_body(tabt_hbm, tail_hbm, lin_hbm, tin, tout0, tout1,
              si0, si1, so0, so1):
    wid = lax.axis_index("s") * NC + lax.axis_index("c")
    touts = (tout0, tout1)
    sis = (si0, si1)
    sos = (so0, so1)

    def issue_in(k, g):
        voc0 = g * GV
        for fg in range(8):
            pltpu.async_copy(
                tabt_hbm.at[pl.ds(fg * 8, 8), pl.ds(voc0, GV)],
                tin.at[k, fg], sis[k])

    def drain_in(k, g):
        voc0 = g * GV
        for fg in range(8):
            pltpu.make_async_copy(
                tabt_hbm.at[pl.ds(fg * 8, 8), pl.ds(voc0, GV)],
                tin.at[k, fg], sis[k]).wait()

    def issue_out(k, g):
        pltpu.async_copy(touts[k], lin_hbm.at[pl.ds(g * GV * D, GV * D)],
                         sos[k])

    def drain_out(k, g):
        pltpu.make_async_copy(
            touts[k], lin_hbm.at[pl.ds(g * GV * D, GV * D)], sos[k]).wait()

    lanes = lax.iota(jnp.int32, 16)

    def transpose_group(k):
        ms = [(lanes + j) & 15 for j in range(16)]
        ihis = [m >> 3 for m in ms]
        ilos = [m & 7 for m in ms]
        sps = [lanes * D + m for m in ms]

        @plsc.parallel_loop(0, GV, step=16)
        def _(v0):
            iv = lanes + v0
            for f0 in range(0, D, 16):
                for j in range(16):
                    v = plsc.load_gather(
                        tin.at[k], [ihis[j] + (f0 >> 3), ilos[j], iv])
                    plsc.store_scatter(
                        touts[k], [sps[j] + (v0 * D + f0)], v)

    def my_group(w, j):
        return w * GPT + j

    issue_in(0, my_group(wid, 0))
    issue_in(1, my_group(wid, 1))

    @pl.loop(0, GPT, step=2)
    def _(j):
        for k in range(2):
            g = my_group(wid, j + k)
            drain_in(k, g)

            @pl.when(j + k >= 2)
            def _():
                drain_out(k, my_group(wid, j + k - 2))

            transpose_group(k)
            issue_out(k, g)

            @pl.when(j + k + 2 < GPT)
            def _():
                issue_in(k, my_group(wid, j + k + 2))

    drain_out(0, my_group(wid, GPT - 2))
    drain_out(1, my_group(wid, GPT - 1))

    @pl.when(wid < NXG)
    def _():
        g = NW * GPT + wid
        issue_in(0, g)
        drain_in(0, g)
        transpose_group(0)
        issue_out(0, g)
        drain_out(0, g)

    @pl.when(wid == NXG)
    def _():
        pltpu.async_copy(tail_hbm, tout0.at[pl.ds(0, VTAIL * D)], si0)
        pltpu.make_async_copy(
            tail_hbm, tout0.at[pl.ds(0, VTAIL * D)], si0).wait()
        pltpu.async_copy(tout0.at[pl.ds(0, VTAIL * D)],
                         lin_hbm.at[pl.ds((V - VTAIL) * D, VTAIL * D)], so0)
        pltpu.make_async_copy(
            tout0.at[pl.ds(0, VTAIL * D)],
            lin_hbm.at[pl.ds((V - VTAIL) * D, VTAIL * D)], so0).wait()


def _tile_body(x_hbm, tab_hbm, out_hbm, x_v, bufs, out_v, *sems):
    wid = lax.axis_index("s") * NC + lax.axis_index("c")
    base = wid * RPT

    pltpu.sync_copy(x_hbm.at[pl.ds(base * L, RPT * L)], x_v)

    def issue(k, r):
        pltpu.async_copy(
            tab_hbm.at[x_v.at[pl.ds(r * L, L)]], bufs.at[k], sems[k])

    def drain(k, r):
        pltpu.make_async_copy(
            tab_hbm.at[x_v.at[pl.ds(r * L, L)]], bufs.at[k], sems[k]).wait()

    for k in range(NBUF):
        issue(k, k)

    lanes = lax.iota(jnp.int32, 16)

    @pl.loop(0, RPT, step=NBUF)
    def _(g):
        for k in range(NBUF):
            r = g + k
            drain(k, r)

            cnt = jnp.zeros((16,), jnp.int32)
            for c in range(NFV):
                v = x_v[pl.ds(r * L + c * 16, 16)]
                cnt = cnt + plsc.all_reduce_population_count(v != 0)
            vt = x_v[pl.ds(r * L + 184, 16)]
            cnt = cnt + plsc.all_reduce_population_count(
                (vt != 0) & (lanes >= 8))
            denom = jnp.maximum(cnt.astype(jnp.float32),
                                jnp.full((16,), 1.0, jnp.float32))

            zero = jnp.zeros((16,), jnp.float32)

            @plsc.parallel_loop(0, L, unroll=8, carry=(zero,) * 4)
            def acc(l, a):
                return tuple(a[d] + bufs[k, l, pl.ds(d * 16, 16)]
                             for d in range(4))

            for d in range(4):
                out_v[r, pl.ds(d * 16, 16)] = acc[d] / denom

            @pl.when(r + NBUF < RPT)
            def _():
                issue(k, r + NBUF)

    pltpu.sync_copy(out_v, out_hbm.at[pl.ds(base, RPT)])


@jax.jit
def _run(x_flat, table):
    mesh = plsc.VectorSubcoreMesh(core_axis_name="c", subcore_axis_name="s")

    tabt = jnp.transpose(table)
    tail = jnp.reshape(table[V - VTAIL:], (-1,))

    fmt_kernel = functools.partial(
        pl.kernel,
        out_type=jax.ShapeDtypeStruct((V * D,), jnp.float32),
        mesh=mesh,
        compiler_params=pltpu.CompilerParams(use_tc_tiling_on_sc=True,
                                             needs_layout_passes=False),
        scratch_types=[
            pltpu.VMEM((2, 8, 8, GV), jnp.float32),
            pltpu.VMEM((GV * D,), jnp.float32),
            pltpu.VMEM((GV * D,), jnp.float32),
        ] + [pltpu.SemaphoreType.DMA] * 4,
    )(_fmt_body)
    tab_lin = fmt_kernel(tabt, tail)

    grid_kernel = functools.partial(
        pl.kernel,
        out_type=jax.ShapeDtypeStruct((B, D), jnp.float32),
        mesh=mesh,
        compiler_params=pltpu.CompilerParams(use_tc_tiling_on_sc=False,
                                             needs_layout_passes=False),
        scratch_types=[
            pltpu.VMEM((RPT * L,), jnp.int32),
            pltpu.VMEM((NBUF, L, D), jnp.float32),
            pltpu.VMEM((RPT, D), jnp.float32),
        ] + [pltpu.SemaphoreType.DMA] * NBUF,
    )(_tile_body)
    return grid_kernel(x_flat, jnp.reshape(tab_lin, (V, D)))


def kernel(x, table):
    return _run(jnp.reshape(x.astype(jnp.int32), (-1,)), table)


# speedup vs baseline: 2.7665x; 1.1651x over previous
import functools

import jax
import jax.numpy as jnp
from jax import lax
from jax.experimental import pallas as pl
from jax.experimental.pallas import tpu as pltpu
from jax.experimental.pallas import tpu_sc as plsc

B = 4096
L = 200
D = 64
V = 1000000
NC = 2
NS = 16
NW = NC * NS
RPT = B // NW
NBUF = 4
NFV = 192 // 16

GV = 128
GPT = (V // GV) // NW
NXG = (V // GV) - GPT * NW
VTAIL = V - (V // GV) * GV


def _fmt_body(tabt_hbm, tail_hbm, lin_hbm, tin, tout0, tout1, tout2, tout3,
              si0, si1, si2, si3, so0, so1, so2, so3):
    wid = lax.axis_index("s") * NC + lax.axis_index("c")
    touts = (tout0, tout1, tout2, tout3)
    sis = (si0, si1, si2, si3)
    sos = (so0, so1, so2, so3)

    def issue_in(k, g):
        voc0 = g * GV
        for fg in range(8):
            pltpu.async_copy(
                tabt_hbm.at[pl.ds(fg * 8, 8), pl.ds(voc0, GV)],
                tin.at[k, fg], sis[k])

    def drain_in(k, g):
        voc0 = g * GV
        for fg in range(8):
            pltpu.make_async_copy(
                tabt_hbm.at[pl.ds(fg * 8, 8), pl.ds(voc0, GV)],
                tin.at[k, fg], sis[k]).wait()

    def issue_out(k, g):
        pltpu.async_copy(touts[k], lin_hbm.at[pl.ds(g * GV * D, GV * D)],
                         sos[k])

    def drain_out(k, g):
        pltpu.make_async_copy(
            touts[k], lin_hbm.at[pl.ds(g * GV * D, GV * D)], sos[k]).wait()

    lanes = lax.iota(jnp.int32, 16)

    def transpose_group(k):
        ms = [(lanes + j) & 15 for j in range(16)]
        ihis = [m >> 3 for m in ms]
        ilos = [m & 7 for m in ms]
        sps = [lanes * D + m for m in ms]

        @plsc.parallel_loop(0, GV, step=16)
        def _(v0):
            iv = lanes + v0
            for f0 in range(0, D, 16):
                for j in range(16):
                    v = plsc.load_gather(
                        tin.at[k], [ihis[j] + (f0 >> 3), ilos[j], iv])
                    plsc.store_scatter(
                        touts[k], [sps[j] + (v0 * D + f0)], v)

    def my_group(w, j):
        return w * GPT + j

    for k in range(4):
        issue_in(k, my_group(wid, k))

    @pl.loop(0, GPT, step=4)
    def _(j):
        for k in range(4):
            g = my_group(wid, j + k)
            drain_in(k, g)

            @pl.when(j + k >= 4)
            def _():
                drain_out(k, my_group(wid, j + k - 4))

            transpose_group(k)
            issue_out(k, g)

            @pl.when(j + k + 4 < GPT)
            def _():
                issue_in(k, my_group(wid, j + k + 4))

    for k in range(4):
        drain_out(k, my_group(wid, GPT - 4 + k))

    @pl.when(wid < NXG)
    def _():
        g = NW * GPT + wid
        issue_in(0, g)
        drain_in(0, g)
        transpose_group(0)
        issue_out(0, g)
        drain_out(0, g)

    @pl.when(wid == NXG)
    def _():
        pltpu.async_copy(tail_hbm, tout0.at[pl.ds(0, VTAIL * D)], si0)
        pltpu.make_async_copy(
            tail_hbm, tout0.at[pl.ds(0, VTAIL * D)], si0).wait()
        pltpu.async_copy(tout0.at[pl.ds(0, VTAIL * D)],
                         lin_hbm.at[pl.ds((V - VTAIL) * D, VTAIL * D)], so0)
        pltpu.make_async_copy(
            tout0.at[pl.ds(0, VTAIL * D)],
            lin_hbm.at[pl.ds((V - VTAIL) * D, VTAIL * D)], so0).wait()


def _tile_body(x_hbm, tab_hbm, out_hbm, x_v, bufs, out_v, *sems):
    wid = lax.axis_index("s") * NC + lax.axis_index("c")
    base = wid * RPT

    pltpu.sync_copy(x_hbm.at[pl.ds(base * L, RPT * L)], x_v)

    def issue(k, r):
        pltpu.async_copy(
            tab_hbm.at[x_v.at[pl.ds(r * L, L)]], bufs.at[k], sems[k])

    def drain(k, r):
        pltpu.make_async_copy(
            tab_hbm.at[x_v.at[pl.ds(r * L, L)]], bufs.at[k], sems[k]).wait()

    for k in range(NBUF):
        issue(k, k)

    lanes = lax.iota(jnp.int32, 16)

    @pl.loop(0, RPT, step=NBUF)
    def _(g):
        for k in range(NBUF):
            r = g + k
            drain(k, r)

            cnt = jnp.zeros((16,), jnp.int32)
            for c in range(NFV):
                v = x_v[pl.ds(r * L + c * 16, 16)]
                cnt = cnt + plsc.all_reduce_population_count(v != 0)
            vt = x_v[pl.ds(r * L + 184, 16)]
            cnt = cnt + plsc.all_reduce_population_count(
                (vt != 0) & (lanes >= 8))
            denom = jnp.maximum(cnt.astype(jnp.float32),
                                jnp.full((16,), 1.0, jnp.float32))

            zero = jnp.zeros((16,), jnp.float32)

            @plsc.parallel_loop(0, L, unroll=8, carry=(zero,) * 4)
            def acc(l, a):
                return tuple(a[d] + bufs[k, l, pl.ds(d * 16, 16)]
                             for d in range(4))

            for d in range(4):
                out_v[r, pl.ds(d * 16, 16)] = acc[d] / denom

            @pl.when(r + NBUF < RPT)
            def _():
                issue(k, r + NBUF)

    pltpu.sync_copy(out_v, out_hbm.at[pl.ds(base, RPT)])


@jax.jit
def _run(x_flat, table):
    mesh = plsc.VectorSubcoreMesh(core_axis_name="c", subcore_axis_name="s")

    tabt = jnp.transpose(table)
    tail = jnp.reshape(table[V - VTAIL:], (-1,))

    fmt_kernel = functools.partial(
        pl.kernel,
        out_type=jax.ShapeDtypeStruct((V * D,), jnp.float32),
        mesh=mesh,
        compiler_params=pltpu.CompilerParams(use_tc_tiling_on_sc=True,
                                             needs_layout_passes=False),
        scratch_types=[
            pltpu.VMEM((4, 8, 8, GV), jnp.float32),
            pltpu.VMEM((GV * D,), jnp.float32),
            pltpu.VMEM((GV * D,), jnp.float32),
            pltpu.VMEM((GV * D,), jnp.float32),
            pltpu.VMEM((GV * D,), jnp.float32),
        ] + [pltpu.SemaphoreType.DMA] * 8,
    )(_fmt_body)
    tab_lin = fmt_kernel(tabt, tail)

    grid_kernel = functools.partial(
        pl.kernel,
        out_type=jax.ShapeDtypeStruct((B, D), jnp.float32),
        mesh=mesh,
        compiler_params=pltpu.CompilerParams(use_tc_tiling_on_sc=False,
                                             needs_layout_passes=False),
        scratch_types=[
            pltpu.VMEM((RPT * L,), jnp.int32),
            pltpu.VMEM((NBUF, L, D), jnp.float32),
            pltpu.VMEM((RPT, D), jnp.float32),
        ] + [pltpu.SemaphoreType.DMA] * NBUF,
    )(_tile_body)
    return grid_kernel(x_flat, jnp.reshape(tab_lin, (V, D)))


def kernel(x, table):
    return _run(jnp.reshape(x.astype(jnp.int32), (-1,)), table)
